# baseline (speedup 1.0000x reference)
import functools

import jax
import jax.numpy as jnp
from jax import lax
from jax.experimental import pallas as pl
from jax.experimental.pallas import tpu as pltpu
from jax.experimental.pallas import tpu_sc as plsc

NC, NS = 2, 16
NW = NC * NS
GB = 128


def _make_kernel(B, L, DC, K, ED, VOCAB):
    D_OUT = DC + K * ED
    BC = B // 128
    BCH = BC // 2
    DH = D_OUT // 8
    CH = DC // 8
    LPT = L // (NW // K)
    mesh = plsc.VectorSubcoreMesh(core_axis_name="c", subcore_axis_name="s")

    @functools.partial(
        pl.kernel,
        out_type=jax.ShapeDtypeStruct((L, DH, BC, 8, 128), jnp.float32),
        mesh=mesh,
        compiler_params=pltpu.CompilerParams(
            use_tc_tiling_on_sc=False, needs_layout_passes=False),
        scratch_types=[
            pltpu.VMEM((2, BC, 128), jnp.int32),
            pltpu.VMEM((2, BCH, 128, ED), jnp.float32),
            pltpu.SemaphoreType.DMA,
            pltpu.SemaphoreType.DMA,
            pltpu.SemaphoreType.DMA,
            pltpu.SemaphoreType.DMA,
            pltpu.SemaphoreType.DMA,
            pltpu.SemaphoreType.DMA,
        ],
    )
    def k(tab_hbm, cat_hbm, cont_hbm, out_hbm,
          idx_v, rows_v, isem0, isem1, gsem, osem0, osem1, csem):
        isem = (isem0, isem1)
        osem = (osem0, osem1)
        wid = lax.axis_index("c") * NS + lax.axis_index("s")
        kk = wid // (NW // K)
        lq = wid % (NW // K)
        l0 = lq * LPT

        def fire_cont(l):
            pltpu.async_copy(cont_hbm.at[l], out_hbm.at[l, pl.ds(0, CH)], csem)

        n_my_cont = (L - 1 - wid) // NW + 1
        def cont_body(i, _):
            fire_cont(wid + i * NW)
            return ()
        lax.fori_loop(0, n_my_cont, cont_body, (), unroll=False)

        def start_idx(i, b):
            pltpu.async_copy(
                cat_hbm.at[l0 + i, :, kk], idx_v.at[b], isem[b])

        def wait_idx(b):
            pltpu.make_async_copy(
                cat_hbm.at[0, :, 0], idx_v.at[b], isem[b]).wait()

        def drain_outs(b):
            for e in range(ED):
                d = DC + kk * ED + e
                pltpu.make_async_copy(
                    rows_v.at[b, :, :, e],
                    out_hbm.at[0, d // 8, pl.ds(0, BCH), d % 8],
                    osem[b]).wait()

        start_idx(0, 0)

        def one_job(i, ib):
            wait_idx(ib)

            @pl.when(i + 1 < LPT)
            def _():
                start_idx(i + 1, 1 - ib)

            l = l0 + i
            for h in range(2):
                b = h
                @pl.when(i > 0)
                def _():
                    drain_outs(b)

                copies = []
                for c in range(BCH):
                    copies.append(pltpu.async_copy(
                        tab_hbm.at[kk].at[idx_v.at[ib, h * BCH + c]],
                        rows_v.at[b, c], gsem))
                for cp in copies:
                    cp.wait()
                for e in range(ED):
                    d = DC + kk * ED + e
                    pltpu.async_copy(
                        rows_v.at[b, :, :, e],
                        out_hbm.at[l, d // 8, pl.ds(h * BCH, BCH), d % 8],
                        osem[b])

        def pair_body(g, _):
            one_job(2 * g, 0)
            one_job(2 * g + 1, 1)
            return ()

        lax.fori_loop(0, LPT // 2, pair_body, (), unroll=False)
        drain_outs(0)
        drain_outs(1)

        def cont_drain(i, _):
            pltpu.make_async_copy(
                cont_hbm.at[0], out_hbm.at[0, pl.ds(0, CH)], csem).wait()
            return ()
        lax.fori_loop(0, n_my_cont, cont_drain, (), unroll=False)

    return k


def kernel(past_exo_cont, past_exo_cat, tables, B, L):
    del B, L
    K, VOCAB, ED = tables.shape
    B, L, DC = past_exo_cont.shape
    cat4 = past_exo_cat.astype(jnp.int32).reshape(
        B // 128, 128, L, K).transpose(2, 0, 3, 1)
    cont5 = past_exo_cont.reshape(
        B // 128, 128, L, DC // 8, 8).transpose(2, 3, 0, 4, 1)
    out5 = _make_kernel(B, L, DC, K, ED, VOCAB)(tables, cat4, cont5)
    out = out5.transpose(2, 4, 0, 1, 3).reshape(B, L, DC + K * ED)
    return out


# speedup vs baseline: 143.4040x; 143.4040x over previous
import functools

import jax
import jax.numpy as jnp
from jax import lax
from jax.experimental import pallas as pl
from jax.experimental.pallas import tpu as pltpu
from jax.experimental.pallas import tpu_sc as plsc

NC, NS = 2, 16
NW = NC * NS
LANES = 16


def _make_kernel(B, L, DC, K, ED, VC):
    D_OUT = DC + K * ED
    BC = B // 128
    DH = D_OUT // 8
    CH = DC // 8
    EPT = K * ED // NW
    mesh = plsc.VectorSubcoreMesh(core_axis_name="c", subcore_axis_name="s")

    @functools.partial(
        pl.kernel,
        out_type=jax.ShapeDtypeStruct((L, DH, BC, 8, 128), jnp.float32),
        mesh=mesh,
        compiler_params=pltpu.CompilerParams(
            use_tc_tiling_on_sc=False, needs_layout_passes=False),
        scratch_types=[
            pltpu.VMEM((VC, 128), jnp.float32),
            pltpu.VMEM((2, BC, 128), jnp.int32),
            pltpu.VMEM((4, BC, 128), jnp.float32),
            pltpu.SemaphoreType.DMA,
            pltpu.SemaphoreType.DMA,
            pltpu.SemaphoreType.DMA,
            pltpu.SemaphoreType.DMA,
            pltpu.SemaphoreType.DMA,
            pltpu.SemaphoreType.DMA,
            pltpu.SemaphoreType.DMA,
            pltpu.SemaphoreType.DMA,
        ],
    )
    def k(tab_hbm, cat_hbm, cont_hbm, out_hbm,
          trow_v, idx_v, orow_v, tsem,
          isem0, isem1, osem0, osem1, osem2, osem3, csem):
        isem = (isem0, isem1)
        osem = (osem0, osem1, osem2, osem3)
        wid = lax.axis_index("c") * NS + lax.axis_index("s")
        kk = wid // (NW // K)
        q = wid % (NW // K)

        def fire_cont(l):
            pltpu.async_copy(cont_hbm.at[l], out_hbm.at[l, pl.ds(0, CH)], csem)

        def start_idx(l, b):
            pltpu.async_copy(cat_hbm.at[l, :, kk], idx_v.at[b], isem[b])

        def wait_idx(b):
            pltpu.make_async_copy(
                cat_hbm.at[0, :, 0], idx_v.at[b], isem[b]).wait()

        n_my_cont = (L - 1 - wid) // NW + 1
        def cont_body(i, _):
            fire_cont(wid + i * NW)
            return ()
        lax.fori_loop(0, n_my_cont, cont_body, (), unroll=False)

        for j in range(EPT):
            e = q * EPT + j
            eh_t, el_t = e // 8, e % 8
            d = DC + kk * ED + e
            dh, dl = d // 8, d % 8

            cp_t = pltpu.async_copy(
                tab_hbm.at[kk, eh_t, :, el_t], trow_v, tsem)
            for p in range(2):
                start_idx(p, p)
            cp_t.wait()

            def compute_q(b_, o_):
                def bc_body(c, _):
                    vs = [idx_v[b_, c, pl.ds(s * LANES, LANES)]
                          for s in range(8)]
                    his = [lax.shift_right_logical(v, 7) for v in vs]
                    los = [lax.bitwise_and(v, 127) for v in vs]
                    gs = [plsc.load_gather(trow_v, [hi, lo])
                          for hi, lo in zip(his, los)]
                    for s, g in enumerate(gs):
                        orow_v[o_, c, pl.ds(s * LANES, LANES)] = g
                    return ()
                lax.fori_loop(0, BC, bc_body, (), unroll=2)

            def quad_body(h, _):
                for p in range(4):
                    l = 4 * h + p
                    ip = p % 2

                    wait_idx(ip)

                    @pl.when(l >= 4)
                    def _():
                        pltpu.make_async_copy(
                            orow_v.at[p],
                            out_hbm.at[0, dh, :, dl], osem[p]).wait()

                    compute_q(ip, p)

                    @pl.when(l + 2 < L)
                    def _():
                        start_idx(l + 2, ip)

                    pltpu.async_copy(
                        orow_v.at[p], out_hbm.at[l, dh, :, dl], osem[p])
                return ()

            lax.fori_loop(0, L // 4, quad_body, (), unroll=False)
            for o in range(4):
                pltpu.make_async_copy(
                    orow_v.at[o], out_hbm.at[0, dh, :, dl], osem[o]).wait()

        def cont_drain(i, _):
            pltpu.make_async_copy(
                cont_hbm.at[0], out_hbm.at[0, pl.ds(0, CH)], csem).wait()
            return ()
        lax.fori_loop(0, n_my_cont, cont_drain, (), unroll=False)

    return k


def kernel(past_exo_cont, past_exo_cat, tables, B, L):
    del B, L
    K, VOCAB, ED = tables.shape
    B, L, DC = past_exo_cont.shape
    VPAD = -VOCAB % 128
    VC = (VOCAB + VPAD) // 128
    cat4 = past_exo_cat.astype(jnp.int32).reshape(
        B // 128, 128, L, K).transpose(2, 0, 3, 1)
    cont5 = past_exo_cont.reshape(
        B // 128, 128, L, DC // 8, 8).transpose(2, 3, 0, 4, 1)
    tab5 = jnp.pad(tables, ((0, 0), (0, VPAD), (0, 0))).reshape(
        K, VC, 128, ED // 8, 8).transpose(0, 3, 1, 4, 2)
    out5 = _make_kernel(B, L, DC, K, ED, VC)(tab5, cat4, cont5)
    out = out5.transpose(2, 4, 0, 1, 3).reshape(B, L, DC + K * ED)
    return out
